# Initial kernel scaffold; baseline (speedup 1.0000x reference)
import jax
import jax.numpy as jnp
from jax.experimental import pallas as pl


def kernel(tokens, table):
    raise NotImplementedError("write your pallas kernel here")


# speedup vs baseline: 1.8793x; 1.8793x over previous
import functools

import jax
import jax.numpy as jnp
from jax import lax
from jax.experimental import pallas as pl
from jax.experimental.pallas import tpu as pltpu
from jax.experimental.pallas import tpu_sc as plsc

_NUM_CORES = 2
_NUM_SUBCORES = 16
_NUM_WORKERS = _NUM_CORES * _NUM_SUBCORES

_HIDDEN = 64
_TOKENS_PER_BIN = 20
_GROUP_BINS = 32
_GROUP_ROWS = _GROUP_BINS * _TOKENS_PER_BIN
_IDX_MINOR = 128
_IDX_CHUNKS = _GROUP_ROWS // _IDX_MINOR


def _make_kernel(num_bins):
  assert num_bins % (_NUM_WORKERS * _GROUP_BINS) == 0
  bins_per_worker = num_bins // _NUM_WORKERS
  num_groups = bins_per_worker // _GROUP_BINS
  tok_rows_per_group = _GROUP_ROWS // _IDX_MINOR
  h_chunks = _HIDDEN // 16

  mesh = plsc.VectorSubcoreMesh(
      core_axis_name="c",
      subcore_axis_name="s",
      num_cores=_NUM_CORES,
      num_subcores=_NUM_SUBCORES,
  )

  @functools.partial(
      pl.kernel,
      mesh=mesh,
      compiler_params=pltpu.CompilerParams(
          needs_layout_passes=False, use_tc_tiling_on_sc=False),
      out_type=jax.ShapeDtypeStruct((num_bins * _HIDDEN, ), jnp.float32),
      scratch_types=[
          pltpu.VMEM((_GROUP_ROWS, ), jnp.int32),
          pltpu.VMEM((_GROUP_ROWS, _HIDDEN), jnp.float32),
          pltpu.VMEM((_GROUP_BINS * _HIDDEN, ), jnp.float32),
          pltpu.SemaphoreType.DMA,
      ],
  )
  def embed(tok_flat_hbm, table_hbm, out_hbm, tokf_v, rows_v, out_v, sem):
    wid = lax.axis_index("s") * _NUM_CORES + lax.axis_index("c")
    worker_bin0 = wid * bins_per_worker
    lane = lax.iota(jnp.int32, 16)

    def group_body(g, carry):
      bin0 = worker_bin0 + g * _GROUP_BINS
      pltpu.sync_copy(
          tok_flat_hbm.at[pl.ds(bin0 * _TOKENS_PER_BIN, _GROUP_ROWS)],
          tokf_v,
      )
      copies = [
          pltpu.async_copy(
              table_hbm.at[tokf_v.at[pl.ds(j * _IDX_MINOR, _IDX_MINOR)]],
              rows_v.at[pl.ds(j * _IDX_MINOR, _IDX_MINOR)],
              sem,
          )
          for j in range(_IDX_CHUNKS)
      ]
      for cp in copies:
        cp.wait()

      def bin_body(b, carry2):
        row0 = b * _TOKENS_PER_BIN
        acc = [rows_v[row0, pl.ds(h * 16, 16)] for h in range(h_chunks)]
        for l in range(1, _TOKENS_PER_BIN):
          r = row0 + l
          for h in range(h_chunks):
            acc[h] = acc[h] + rows_v[r, pl.ds(h * 16, 16)]
        for h in range(h_chunks):
          out_v[pl.ds(b * _HIDDEN + h * 16, 16)] = acc[h]
        return carry2

      lax.fori_loop(0, _GROUP_BINS, bin_body, 0)

      for half in range(_GROUP_BINS // 16):
        binv = lane + half * 16
        cnt = jnp.zeros((16,), jnp.int32)
        for l in range(_TOKENS_PER_BIN):
          tv = plsc.load_gather(tokf_v, [binv * _TOKENS_PER_BIN + l])
          cnt = cnt + jnp.where(tv != 0, jnp.int32(1), jnp.int32(0))
        inv = 1.0 / jnp.maximum(cnt, 1).astype(jnp.float32)

        def col_body(d, carry3, binv=binv, inv=inv):
          idx = binv * _HIDDEN + d
          col = plsc.load_gather(out_v, [idx])
          plsc.store_scatter(out_v, [idx], col * inv)
          return carry3

        lax.fori_loop(0, _HIDDEN, col_body, 0)

      pltpu.sync_copy(
          out_v, out_hbm.at[pl.ds(bin0 * _HIDDEN, _GROUP_BINS * _HIDDEN)])
      return carry

    lax.fori_loop(0, num_groups, group_body, 0)

  return embed


def kernel(tokens, table):
  assert tokens.ndim == 3 and table.ndim == 2
  batch, feats, tpb = tokens.shape
  assert tpb == _TOKENS_PER_BIN and table.shape[1] == _HIDDEN
  num_bins = batch * feats
  tok_flat = tokens.astype(jnp.int32).reshape(-1)
  out = _make_kernel(num_bins)(tok_flat, table)
  return out.reshape(batch, feats, _HIDDEN)
